# Initial kernel scaffold; baseline (speedup 1.0000x reference)
import jax
import jax.numpy as jnp
from jax.experimental import pallas as pl


def kernel(node_ids, edge_index, client_emb, item_emb, W_self, W_neigh, b):
    raise NotImplementedError("write your pallas kernel here")


# speedup vs baseline: 3.8449x; 3.8449x over previous
import functools

import jax
import jax.numpy as jnp
from jax import lax
from jax.experimental import pallas as pl
from jax.experimental.pallas import tpu as pltpu
from jax.experimental.pallas import tpu_sc as plsc

N_USERS = 100000
N_ITEMS = 100000
D = 128
H = 128
N_SUB = 10000
N_EDGES = 320000

NC = 2
NS = 16
NW = NC * NS

R = 10240
RB = R // 128

A_CHUNK = 80
C_CHUNK = 80
C_STEPS = N_EDGES // NW // C_CHUNK

_mesh = plsc.VectorSubcoreMesh(
    core_axis_name="c", subcore_axis_name="s", num_cores=NC, num_subcores=NS
)


def _gather_body(idxc_hbm, idxi_hbm, cemb_hbm, iemb_hbm, xc_hbm, xi_hbm,
                 idxc_v, idxi_v, rowsc_v, rowsi_v, sem):
    cid = lax.axis_index("c")
    sid = lax.axis_index("s")
    wid = cid * NS + sid
    base = wid * (R // NW)

    @pl.loop(0, (R // NW) // A_CHUNK)
    def _(j):
        off = base + j * A_CHUNK
        pltpu.sync_copy(idxc_hbm.at[pl.ds(off, A_CHUNK)], idxc_v)
        pltpu.sync_copy(idxi_hbm.at[pl.ds(off, A_CHUNK)], idxi_v)
        pltpu.async_copy(cemb_hbm.at[idxc_v], rowsc_v, sem).wait()
        pltpu.async_copy(iemb_hbm.at[idxi_v], rowsi_v, sem).wait()
        pltpu.sync_copy(rowsc_v, xc_hbm.at[pl.ds(off, A_CHUNK)])
        pltpu.sync_copy(rowsi_v, xi_hbm.at[pl.ds(off, A_CHUNK)])


_gather_call = functools.partial(
    pl.kernel,
    out_type=[
        jax.ShapeDtypeStruct((R, D), jnp.float32),
        jax.ShapeDtypeStruct((R, D), jnp.float32),
    ],
    mesh=_mesh,
    scratch_types=[
        pltpu.VMEM((A_CHUNK,), jnp.int32),
        pltpu.VMEM((A_CHUNK,), jnp.int32),
        pltpu.VMEM((A_CHUNK, D), jnp.float32),
        pltpu.VMEM((A_CHUNK, D), jnp.float32),
        pltpu.SemaphoreType.DMA,
    ],
)


def _zero_2d(ref):
    rows, cols = ref.shape

    @pl.loop(0, rows)
    def _(r):
        @pl.loop(0, cols, step=16)
        def _(cc):
            ref.at[r, pl.ds(cc, 16)][...] = jnp.zeros((16,), jnp.float32)


def _agg_body(src_hbm, dst_hbm, xn_hbm, acc_hbm, deg_hbm,
              sidx_v, didx_v, rows_v, dloc_v, lin_v, acc_sh, dacc_sh, sem):
    cid = lax.axis_index("c")
    sid = lax.axis_index("s")
    wid = cid * NS + sid
    rows_per_tile = R // NS

    _zero_2d(rows_v)
    _zero_2d(dloc_v)

    @pl.loop(0, RB, step=16)
    def _(k):
        lin_v.at[pl.ds(k, 16)][...] = lax.iota(jnp.int32, 16) + k

    @pl.loop(0, rows_per_tile // C_CHUNK)
    def _(k):
        pltpu.sync_copy(rows_v, acc_sh.at[pl.ds(sid * rows_per_tile
                                                + k * C_CHUNK, C_CHUNK)])
    @pl.when(sid < RB // 8)
    def _():
        pltpu.sync_copy(rows_v.at[pl.ds(0, 8)],
                        dacc_sh.at[pl.ds(sid * 8, 8)])

    plsc.subcore_barrier()

    base = wid * (N_EDGES // NW)

    @pl.loop(0, C_STEPS)
    def _(j):
        off = base + j * C_CHUNK
        pltpu.sync_copy(src_hbm.at[pl.ds(off, C_CHUNK)], sidx_v)
        pltpu.sync_copy(dst_hbm.at[pl.ds(off, C_CHUNK)], didx_v)
        pltpu.async_copy(xn_hbm.at[sidx_v], rows_v, sem).wait()
        pltpu.sync_copy(rows_v, acc_sh.at[didx_v], add=True)

        @pl.loop(0, C_CHUNK, step=16)
        def _(k):
            d16 = didx_v.at[pl.ds(k, 16)][...]
            plsc.addupdate_scatter(
                dloc_v,
                [lax.shift_right_logical(d16, 7),
                 lax.bitwise_and(d16, 127)],
                jnp.full((16,), 1.0, jnp.float32))

    pltpu.sync_copy(dloc_v, dacc_sh.at[lin_v], add=True)
    plsc.subcore_barrier()

    pltpu.sync_copy(acc_sh.at[pl.ds(sid * rows_per_tile, rows_per_tile)],
                    acc_hbm.at[cid, pl.ds(sid * rows_per_tile, rows_per_tile)])
    @pl.when(sid < RB // 8)
    def _():
        pltpu.sync_copy(dacc_sh.at[pl.ds(sid * 8, 8)],
                        deg_hbm.at[cid, pl.ds(sid * 8, 8)])


_agg_call = functools.partial(
    pl.kernel,
    out_type=[
        jax.ShapeDtypeStruct((NC, R, D), jnp.float32),
        jax.ShapeDtypeStruct((NC, RB, 128), jnp.float32),
    ],
    mesh=_mesh,
    scratch_types=[
        pltpu.VMEM((C_CHUNK,), jnp.int32),
        pltpu.VMEM((C_CHUNK,), jnp.int32),
        pltpu.VMEM((C_CHUNK, D), jnp.float32),
        pltpu.VMEM((RB, 128), jnp.float32),
        pltpu.VMEM((RB,), jnp.int32),
        pltpu.VMEM_SHARED((R, D), jnp.float32),
        pltpu.VMEM_SHARED((RB, 128), jnp.float32),
        pltpu.SemaphoreType.DMA,
    ],
    compiler_params=pltpu.CompilerParams(needs_layout_passes=False),
)


def _matmul_body(ids_ref, xc_ref, xi_ref, ws_ref, wn_ref, xs_ref, xn_ref):
    x = jnp.where(ids_ref[...] < N_USERS, xc_ref[...], xi_ref[...])
    xs_ref[...] = jnp.dot(x, ws_ref[...], preferred_element_type=jnp.float32)
    xn_ref[...] = jnp.dot(x, wn_ref[...], preferred_element_type=jnp.float32)


def _combine_body(xs_ref, acc_ref, deg_ref, b_ref, h_ref):
    agg = acc_ref[0] + acc_ref[1]
    deg = deg_ref[...]
    h_ref[...] = jnp.maximum(
        xs_ref[...] + agg / jnp.maximum(deg, 1.0) + b_ref[...], 0.0)


def kernel(node_ids, edge_index, client_emb, item_emb, W_self, W_neigh, b):
    ids_pad = jnp.zeros((R,), jnp.int32).at[:N_SUB].set(
        node_ids.astype(jnp.int32))
    is_client = ids_pad < N_USERS
    idx_c = jnp.where(is_client, ids_pad, 0)
    idx_i = jnp.where(is_client, 0, ids_pad - N_USERS)
    src = edge_index[0].astype(jnp.int32)
    dst = edge_index[1].astype(jnp.int32)

    xc, xi = _gather_call(_gather_body)(idx_c, idx_i, client_emb, item_emb)

    blk = 1280
    xs, xn = pl.pallas_call(
        _matmul_body,
        grid=(R // blk,),
        in_specs=[
            pl.BlockSpec((blk, 1), lambda i: (i, 0)),
            pl.BlockSpec((blk, D), lambda i: (i, 0)),
            pl.BlockSpec((blk, D), lambda i: (i, 0)),
            pl.BlockSpec((D, H), lambda i: (0, 0)),
            pl.BlockSpec((D, H), lambda i: (0, 0)),
        ],
        out_specs=[
            pl.BlockSpec((blk, H), lambda i: (i, 0)),
            pl.BlockSpec((blk, H), lambda i: (i, 0)),
        ],
        out_shape=[
            jax.ShapeDtypeStruct((R, H), jnp.float32),
            jax.ShapeDtypeStruct((R, H), jnp.float32),
        ],
    )(ids_pad[:, None], xc, xi, W_self, W_neigh)

    accs, degs = _agg_call(_agg_body)(src, dst, xn)

    deg_col = (degs[0] + degs[1]).reshape(R)[:, None]

    cblk = 1000
    h = pl.pallas_call(
        _combine_body,
        grid=(N_SUB // cblk,),
        in_specs=[
            pl.BlockSpec((cblk, H), lambda i: (i, 0)),
            pl.BlockSpec((NC, cblk, H), lambda i: (0, i, 0)),
            pl.BlockSpec((cblk, 1), lambda i: (i, 0)),
            pl.BlockSpec((1, H), lambda i: (0, 0)),
        ],
        out_specs=pl.BlockSpec((cblk, H), lambda i: (i, 0)),
        out_shape=jax.ShapeDtypeStruct((N_SUB, H), jnp.float32),
    )(xs, accs, deg_col, b[None, :])
    return h
